# baseline (device time: 70356 ns/iter reference)
import jax
import jax.numpy as jnp
from jax import lax
from jax.experimental import pallas as pl
from jax.experimental.pallas import tpu as pltpu

N_DEV = 4
BLK = 64


def kernel(x, Wq, K_ext, V_ext, Wo):
    B, Sq, E = x.shape
    _, Skv, Hq, Dh = K_ext.shape
    Dq = Wq.shape[1]

    def body(x_ref, wq_ref, k_ref, v_ref, wo_ref, out_ref,
             ctx_ref, send_sems, recv_sem, ready_sem):
        my = lax.axis_index("i")
        left = (my + N_DEV - 1) % N_DEV
        right = (my + 1) % N_DEV

        barrier_sem = pltpu.get_barrier_semaphore()
        for nbr in (left, right):
            pl.semaphore_signal(
                barrier_sem, inc=1,
                device_id=(nbr,), device_id_type=pl.DeviceIdType.MESH,
            )
        pl.semaphore_wait(barrier_sem, 2)

        @pl.when(jnp.logical_or(my == 1, my == 3))
        def _():
            pl.semaphore_signal(
                ready_sem, inc=1,
                device_id=(0,), device_id_type=pl.DeviceIdType.MESH,
            )

        @pl.when(my == 2)
        def _():
            pl.semaphore_signal(
                ready_sem, inc=1,
                device_id=(1,), device_id_type=pl.DeviceIdType.MESH,
            )

        @pl.when(my == 0)
        def _():
            qb = lax.broadcasted_iota(jnp.int32, (Sq, Skv), 0) // BLK
            kb = lax.broadcasted_iota(jnp.int32, (Sq, Skv), 1) // BLK
            mask = kb <= qb
            for b in range(B):
                q_b = jnp.dot(x_ref[b], wq_ref[...],
                              preferred_element_type=jnp.float32)
                for h in range(Hq):
                    q_bh = q_b[:, h * Dh:(h + 1) * Dh]
                    k_bh = k_ref[b, :, h, :]
                    v_bh = v_ref[b, :, h, :]
                    s = lax.dot_general(
                        q_bh, k_bh, (((1,), (1,)), ((), ())),
                        preferred_element_type=jnp.float32,
                    ) * 0.125
                    s = jnp.where(mask, s, -1e9)
                    m = jnp.max(s, axis=-1, keepdims=True)
                    w = jnp.exp(s - m)
                    w = w / jnp.sum(w, axis=-1, keepdims=True)
                    ctx_ref[b, :, h * Dh:(h + 1) * Dh] = jnp.dot(
                        w, v_bh, preferred_element_type=jnp.float32)

        to1 = pltpu.make_async_remote_copy(
            src_ref=ctx_ref, dst_ref=ctx_ref,
            send_sem=send_sems.at[0], recv_sem=recv_sem,
            device_id=(1,), device_id_type=pl.DeviceIdType.MESH,
        )
        to3 = pltpu.make_async_remote_copy(
            src_ref=ctx_ref, dst_ref=ctx_ref,
            send_sem=send_sems.at[1], recv_sem=recv_sem,
            device_id=(3,), device_id_type=pl.DeviceIdType.MESH,
        )
        to2 = pltpu.make_async_remote_copy(
            src_ref=ctx_ref, dst_ref=ctx_ref,
            send_sem=send_sems.at[0], recv_sem=recv_sem,
            device_id=(2,), device_id_type=pl.DeviceIdType.MESH,
        )

        @pl.when(my == 0)
        def _():
            pl.semaphore_wait(ready_sem, 2)
            to1.start()
            to3.start()

        @pl.when(my != 0)
        def _():
            to1.wait_recv()

        @pl.when(my == 1)
        def _():
            pl.semaphore_wait(ready_sem, 1)
            to2.start()

        for b in range(B):
            out_ref[b] = jnp.dot(ctx_ref[b], wo_ref[...],
                                 preferred_element_type=jnp.float32)

        @pl.when(my == 0)
        def _():
            to1.wait_send()
            to3.wait_send()

        @pl.when(my == 1)
        def _():
            to2.wait_send()

    return pl.pallas_call(
        body,
        out_shape=jax.ShapeDtypeStruct((B, Sq, E), jnp.float32),
        in_specs=[pl.BlockSpec(memory_space=pltpu.VMEM)] * 5,
        out_specs=pl.BlockSpec(memory_space=pltpu.VMEM),
        scratch_shapes=[
            pltpu.VMEM((B, Sq, Dq), jnp.float32),
            pltpu.SemaphoreType.DMA((2,)),
            pltpu.SemaphoreType.DMA,
            pltpu.SemaphoreType.REGULAR,
        ],
        compiler_params=pltpu.CompilerParams(collective_id=0),
    )(x, Wq, K_ext, V_ext, Wo)


# device time: 39788 ns/iter; 1.7683x vs baseline; 1.7683x over previous
import jax
import jax.numpy as jnp
from jax import lax
from jax.experimental import pallas as pl
from jax.experimental.pallas import tpu as pltpu

N_DEV = 4
BLK = 64


def kernel(x, Wq, K_ext, V_ext, Wo):
    B, Sq, E = x.shape
    _, Skv, Hq, Dh = K_ext.shape
    Dq = Wq.shape[1]

    def body(x_ref, wq_ref, k_ref, v_ref, wo_ref, out_ref,
             ctx_ref, send_sems, recv_sems, ready_sem):
        my = lax.axis_index("i")
        left = (my + N_DEV - 1) % N_DEV
        right = (my + 1) % N_DEV

        barrier_sem = pltpu.get_barrier_semaphore()
        for nbr in (left, right):
            pl.semaphore_signal(
                barrier_sem, inc=1,
                device_id=(nbr,), device_id_type=pl.DeviceIdType.MESH,
            )
        pl.semaphore_wait(barrier_sem, 2)

        @pl.when(jnp.logical_or(my == 1, my == 3))
        def _():
            pl.semaphore_signal(
                ready_sem, inc=1,
                device_id=(0,), device_id_type=pl.DeviceIdType.MESH,
            )

        @pl.when(my == 2)
        def _():
            pl.semaphore_signal(
                ready_sem, inc=1,
                device_id=(1,), device_id_type=pl.DeviceIdType.MESH,
            )

        def copy_chunk(b, target, send_slot):
            return pltpu.make_async_remote_copy(
                src_ref=ctx_ref.at[b], dst_ref=ctx_ref.at[b],
                send_sem=send_sems.at[send_slot, b], recv_sem=recv_sems.at[b],
                device_id=(target,), device_id_type=pl.DeviceIdType.MESH,
            )

        to1 = [copy_chunk(b, 1, 0) for b in range(B)]
        to3 = [copy_chunk(b, 3, 1) for b in range(B)]
        fwd2 = [copy_chunk(b, 2, 0) for b in range(B)]

        wq_bf = wq_ref[...].astype(jnp.bfloat16)
        qb_blk = lax.broadcasted_iota(jnp.int32, (Sq, Skv), 0) // BLK
        kb_blk = lax.broadcasted_iota(jnp.int32, (Sq, Skv), 1) // BLK
        mask = kb_blk <= qb_blk

        def compute_chunk(b):
            q_b = jnp.dot(x_ref[b].astype(jnp.bfloat16), wq_bf,
                          preferred_element_type=jnp.float32)
            for h in range(Hq):
                q_bh = q_b[:, h * Dh:(h + 1) * Dh].astype(jnp.bfloat16)
                k_bh = k_ref[b, :, h, :].astype(jnp.bfloat16)
                v_bh = v_ref[b, :, h, :].astype(jnp.bfloat16)
                s = lax.dot_general(
                    q_bh, k_bh, (((1,), (1,)), ((), ())),
                    preferred_element_type=jnp.float32,
                ) * 0.125
                s = jnp.where(mask, s, -1e9)
                m = jnp.max(s, axis=-1, keepdims=True)
                w = jnp.exp(s - m)
                w = (w / jnp.sum(w, axis=-1, keepdims=True)).astype(jnp.bfloat16)
                ctx_ref[b, :, h * Dh:(h + 1) * Dh] = jnp.dot(
                    w, v_bh, preferred_element_type=jnp.float32,
                ).astype(jnp.bfloat16)

        @pl.when(my == 0)
        def _():
            compute_chunk(0)
            pl.semaphore_wait(ready_sem, 2)
            to1[0].start()
            to3[0].start()
            compute_chunk(1)
            to1[1].start()
            to3[1].start()

        wo_bf = wo_ref[...].astype(jnp.bfloat16)

        def proj_chunk(b):
            out_ref[b] = jnp.dot(ctx_ref[b], wo_bf,
                                 preferred_element_type=jnp.float32)

        @pl.when(my != 0)
        def _():
            to1[0].wait_recv()

        @pl.when(my == 1)
        def _():
            pl.semaphore_wait(ready_sem, 1)
            fwd2[0].start()

        proj_chunk(0)

        @pl.when(my != 0)
        def _():
            to1[1].wait_recv()

        @pl.when(my == 1)
        def _():
            fwd2[1].start()

        proj_chunk(1)

        @pl.when(my == 0)
        def _():
            for b in range(B):
                to1[b].wait_send()
                to3[b].wait_send()

        @pl.when(my == 1)
        def _():
            for b in range(B):
                fwd2[b].wait_send()

    return pl.pallas_call(
        body,
        out_shape=jax.ShapeDtypeStruct((B, Sq, E), jnp.float32),
        in_specs=[pl.BlockSpec(memory_space=pltpu.VMEM)] * 5,
        out_specs=pl.BlockSpec(memory_space=pltpu.VMEM),
        scratch_shapes=[
            pltpu.VMEM((B, Sq, Dq), jnp.bfloat16),
            pltpu.SemaphoreType.DMA((2, B)),
            pltpu.SemaphoreType.DMA((B,)),
            pltpu.SemaphoreType.REGULAR,
        ],
        compiler_params=pltpu.CompilerParams(collective_id=0),
    )(x, Wq, K_ext, V_ext, Wo)


# device time: 39322 ns/iter; 1.7892x vs baseline; 1.0119x over previous
import jax
import jax.numpy as jnp
from jax import lax
from jax.experimental import pallas as pl
from jax.experimental.pallas import tpu as pltpu

N_DEV = 4
BLK = 64
QCH = 2


def kernel(x, Wq, K_ext, V_ext, Wo):
    B, Sq, E = x.shape
    _, Skv, Hq, Dh = K_ext.shape
    Dq = Wq.shape[1]
    Sh = Sq // QCH
    NCH = B * QCH

    def body(x_ref, wq_ref, k_ref, v_ref, wo_ref, out_ref,
             ctx_ref, send_sems, recv_sems, ready_sem):
        my = lax.axis_index("i")
        left = (my + N_DEV - 1) % N_DEV
        right = (my + 1) % N_DEV

        barrier_sem = pltpu.get_barrier_semaphore()
        for nbr in (left, right):
            pl.semaphore_signal(
                barrier_sem, inc=1,
                device_id=(nbr,), device_id_type=pl.DeviceIdType.MESH,
            )
        pl.semaphore_wait(barrier_sem, 2)

        @pl.when(jnp.logical_or(my == 1, my == 3))
        def _():
            pl.semaphore_signal(
                ready_sem, inc=1,
                device_id=(0,), device_id_type=pl.DeviceIdType.MESH,
            )

        @pl.when(my == 2)
        def _():
            pl.semaphore_signal(
                ready_sem, inc=1,
                device_id=(1,), device_id_type=pl.DeviceIdType.MESH,
            )

        def copy_chunk(c, target, send_slot):
            b, qh = divmod(c, QCH)
            chunk = ctx_ref.at[b, pl.ds(qh * Sh, Sh)]
            return pltpu.make_async_remote_copy(
                src_ref=chunk, dst_ref=chunk,
                send_sem=send_sems.at[send_slot, c], recv_sem=recv_sems.at[c],
                device_id=(target,), device_id_type=pl.DeviceIdType.MESH,
            )

        to1 = [copy_chunk(c, 1, 0) for c in range(NCH)]
        to3 = [copy_chunk(c, 3, 1) for c in range(NCH)]
        fwd2 = [copy_chunk(c, 2, 0) for c in range(NCH)]

        wo_bf = wo_ref[...].astype(jnp.bfloat16)

        def proj_chunk(c):
            b, qh = divmod(c, QCH)
            rows = pl.ds(qh * Sh, Sh)
            out_ref[b, rows] = jnp.dot(ctx_ref[b, rows], wo_bf,
                                       preferred_element_type=jnp.float32)

        @pl.when(my == 0)
        def _():
            wq_bf = (wq_ref[...] * 0.125).astype(jnp.bfloat16)
            masks = []
            for qh in range(QCH):
                nk = (qh + 1) * Sh
                qb = (lax.broadcasted_iota(jnp.int32, (Sh, nk), 0)
                      + qh * Sh) // BLK
                kb = lax.broadcasted_iota(jnp.int32, (Sh, nk), 1) // BLK
                masks.append(kb <= qb)

            def compute_chunk(c):
                b, qh = divmod(c, QCH)
                nk = (qh + 1) * Sh
                rows = pl.ds(qh * Sh, Sh)
                q_c = jnp.dot(x_ref[b, rows].astype(jnp.bfloat16), wq_bf,
                              preferred_element_type=jnp.float32)
                for h in range(Hq):
                    q_bh = q_c[:, h * Dh:(h + 1) * Dh].astype(jnp.bfloat16)
                    k_bh = k_ref[b, pl.ds(0, nk), h, :].astype(jnp.bfloat16)
                    v_bh = v_ref[b, pl.ds(0, nk), h, :].astype(jnp.bfloat16)
                    s = lax.dot_general(
                        q_bh, k_bh, (((1,), (1,)), ((), ())),
                        preferred_element_type=jnp.float32,
                    )
                    w = jnp.exp(jnp.where(masks[qh], s, -1e9))
                    w = (w / jnp.sum(w, axis=-1, keepdims=True)
                         ).astype(jnp.bfloat16)
                    ctx_ref[b, rows, h * Dh:(h + 1) * Dh] = jnp.dot(
                        w, v_bh, preferred_element_type=jnp.float32,
                    ).astype(jnp.bfloat16)

            compute_chunk(0)
            pl.semaphore_wait(ready_sem, 2)
            to1[0].start()
            to3[0].start()
            for c in range(1, NCH):
                compute_chunk(c)
                to1[c].start()
                to3[c].start()
            for c in range(NCH):
                proj_chunk(c)
            for c in range(NCH):
                to1[c].wait_send()
                to3[c].wait_send()

        @pl.when(my == 1)
        def _():
            pl.semaphore_wait(ready_sem, 1)
            for c in range(NCH):
                to1[c].wait_recv()
                fwd2[c].start()
                proj_chunk(c)
            for c in range(NCH):
                fwd2[c].wait_send()

        @pl.when(jnp.logical_or(my == 2, my == 3))
        def _():
            for c in range(NCH):
                to1[c].wait_recv()
                proj_chunk(c)

    return pl.pallas_call(
        body,
        out_shape=jax.ShapeDtypeStruct((B, Sq, E), jnp.float32),
        in_specs=[pl.BlockSpec(memory_space=pltpu.VMEM)] * 5,
        out_specs=pl.BlockSpec(memory_space=pltpu.VMEM),
        scratch_shapes=[
            pltpu.VMEM((B, Sq, Dq), jnp.bfloat16),
            pltpu.SemaphoreType.DMA((2, B * QCH)),
            pltpu.SemaphoreType.DMA((B * QCH,)),
            pltpu.SemaphoreType.REGULAR,
        ],
        compiler_params=pltpu.CompilerParams(collective_id=0),
    )(x, Wq, K_ext, V_ext, Wo)
